# baseline (speedup 1.0000x reference)
import functools

import jax
import jax.numpy as jnp
from jax import lax
from jax.experimental import pallas as pl
from jax.experimental.pallas import tpu as pltpu
from jax.experimental.pallas import tpu_sc as plsc

_N = 10000
_E = 320000
_D = 128
_HEADS = 3
_ALPHA = 0.2

_NC = 2
_NS = 16
_K = 80
_EPW = _E // (_NC * _NS)
_NCH = _EPW // _K
_NPAD = 10240


def _proj_body(h_ref, w_ref, a_ref, ht0_ref, ht1_ref, ht2_ref, s1_ref, s2_ref):
    hb = h_ref[...]
    ht_refs = (ht0_ref, ht1_ref, ht2_ref)
    for h in range(_HEADS):
        w = w_ref[h]
        ht = lax.dot_general(hb, w, (((1,), (1,)), ((), ())),
                             preferred_element_type=jnp.float32)
        ht_refs[h][...] = ht
        s1_ref[h, 0] = jnp.dot(ht, a_ref[h, :_D],
                               preferred_element_type=jnp.float32)
        s2_ref[h, 0] = jnp.dot(ht, a_ref[h, _D:],
                               preferred_element_type=jnp.float32)


def _project(H, W, a):
    bn = 1024
    grid = _NPAD // bn
    out_shape = (
        jax.ShapeDtypeStruct((_NPAD, _D), jnp.float32),
        jax.ShapeDtypeStruct((_NPAD, _D), jnp.float32),
        jax.ShapeDtypeStruct((_NPAD, _D), jnp.float32),
        jax.ShapeDtypeStruct((_HEADS, 1, _NPAD), jnp.float32),
        jax.ShapeDtypeStruct((_HEADS, 1, _NPAD), jnp.float32),
    )
    return pl.pallas_call(
        _proj_body,
        grid=(grid,),
        in_specs=[
            pl.BlockSpec((bn, _D), lambda i: (i, 0)),
            pl.BlockSpec((_HEADS, _D, _D), lambda i: (0, 0, 0)),
            pl.BlockSpec((_HEADS, 2 * _D), lambda i: (0, 0)),
        ],
        out_specs=(
            pl.BlockSpec((bn, _D), lambda i: (i, 0)),
            pl.BlockSpec((bn, _D), lambda i: (i, 0)),
            pl.BlockSpec((bn, _D), lambda i: (i, 0)),
            pl.BlockSpec((_HEADS, 1, bn), lambda i: (0, 0, i)),
            pl.BlockSpec((_HEADS, 1, bn), lambda i: (0, 0, i)),
        ),
        out_shape=out_shape,
    )(H, W, a)


def _score_body(s1h, s2h, srch, dsth, pv_out,
                s1v, s2v, src_all, dst_all, pv_all):
    c = lax.axis_index("c")
    s = lax.axis_index("s")
    base_e = c * (_E // _NC) + s * _EPW
    pltpu.sync_copy(srch.at[pl.ds(base_e, _EPW)], src_all)
    pltpu.sync_copy(dsth.at[pl.ds(base_e, _EPW)], dst_all)
    for h in range(_HEADS):
        pltpu.sync_copy(s1h.at[h, 0], s1v)
        pltpu.sync_copy(s2h.at[h, 0], s2v)

        @plsc.parallel_loop(0, _EPW // 16, unroll=4)
        def _(g):
            sl = pl.ds(g * 16, 16)
            e = (plsc.load_gather(s1v, [src_all[sl]])
                 + plsc.load_gather(s2v, [dst_all[sl]]))
            e = jnp.where(e > 0, e, _ALPHA * e)
            pv_all[sl] = jnp.exp(e)

        pltpu.sync_copy(pv_all, pv_out.at[pl.ds(h * _E + base_e, _EPW)])


def _score(s1, s2, src, dst):
    mesh = plsc.VectorSubcoreMesh(core_axis_name="c", subcore_axis_name="s")
    fn = functools.partial(
        pl.kernel,
        out_type=jax.ShapeDtypeStruct((_HEADS * _E,), jnp.float32),
        mesh=mesh,
        scratch_types=[
            pltpu.VMEM((_NPAD,), jnp.float32),
            pltpu.VMEM((_NPAD,), jnp.float32),
            pltpu.VMEM((_EPW,), jnp.int32),
            pltpu.VMEM((_EPW,), jnp.int32),
            pltpu.VMEM((_EPW,), jnp.float32),
        ],
        compiler_params=pltpu.CompilerParams(needs_layout_passes=False),
    )(_score_body)
    return fn(s1, s2, src, dst)


def _agg_body(ht0, ht1, ht2, srch, dsth, pvh, u_out, d_out,
              u_sh, d_sh, gbuf0, gbuf1, sbuf0, sbuf1,
              srcv0, srcv1, dstv0, dstv1, scv0, scv1,
              pvb0, pvb1, pvs0, pvs1, zvec,
              semio0, semio1, semg0, semg1, sems0, sems1):
    gbufs = (gbuf0, gbuf1)
    sbufs = (sbuf0, sbuf1)
    srcvs = (srcv0, srcv1)
    dstvs = (dstv0, dstv1)
    scvs = (scv0, scv1)
    pvbs = (pvb0, pvb1)
    pvss = (pvs0, pvs1)
    semio = (semio0, semio1)
    semg = (semg0, semg1)
    sems = (sems0, sems1)
    c = lax.axis_index("c")
    s = lax.axis_index("s")
    z16 = jnp.zeros((16,), jnp.float32)
    base_e = c * (_E // _NC) + s * _EPW

    def _zv(i, carry):
        zvec[pl.ds(i * 16, 16)] = z16
        return carry
    lax.fori_loop(0, zvec.shape[0] // 16, _zv, 0)

    ht_hbms = (ht0, ht1, ht2)
    for h in range(_HEADS):
        ht_h = ht_hbms[h]

        def _zg(i, carry):
            for s8 in range(8):
                gbuf0[i, pl.ds(s8 * 16, 16)] = z16
            return carry
        lax.fori_loop(0, _K, _zg, 0)
        for j in range(8):
            pltpu.sync_copy(gbuf0, u_sh.at[pl.ds(s * 640 + j * _K, _K)])

        @pl.when(s < 10)
        def _():
            pltpu.sync_copy(zvec, d_sh.at[pl.ds(s * 1024, 1024)])

        def _fire_io(b, ch):
            off = base_e + ch * _K
            pltpu.async_copy(srch.at[pl.ds(off, _K)], srcvs[b], semio[b])
            pltpu.async_copy(dsth.at[pl.ds(off, _K)], dstvs[b], semio[b])
            pltpu.async_copy(pvh.at[pl.ds(h * _E + off, _K)], pvbs[b],
                             semio[b])

        def _wait_io(b):
            pltpu.make_async_copy(srch.at[pl.ds(0, _K)], srcvs[b],
                                  semio[b]).wait()
            pltpu.make_async_copy(dsth.at[pl.ds(0, _K)], dstvs[b],
                                  semio[b]).wait()
            pltpu.make_async_copy(pvh.at[pl.ds(0, _K)], pvbs[b],
                                  semio[b]).wait()

        def _fire_g(b):
            pltpu.async_copy(ht_h.at[dstvs[b]], gbufs[b], semg[b])

        def _wait_g(b):
            pltpu.make_async_copy(ht_h.at[dstvs[b]], gbufs[b],
                                  semg[b]).wait()

        def _scale(b):
            gbuf, sbuf, pvb = gbufs[b], sbufs[b], pvss[b]

            @plsc.parallel_loop(0, _K, unroll=4)
            def _(i):
                pb = plsc.load_gather(pvb, [jnp.full((16,), i, jnp.int32)])
                for s8 in range(8):
                    sl = pl.ds(s8 * 16, 16)
                    sbuf[i, sl] = gbuf[i, sl] * pb

        def _copy_scatter_idx(b):
            for g in range(_K // 16):
                sl = pl.ds(g * 16, 16)
                scvs[b][sl] = srcvs[b][sl]
                pvss[b][sl] = pvbs[b][sl]

        def _fire_scat(b):
            pltpu.async_copy(sbufs[b], u_sh.at[scvs[b]], sems[b], add=True)
            pltpu.async_copy(pvss[b], d_sh.at[scvs[b]], sems[b], add=True)

        def _wait_scat(b):
            pltpu.make_async_copy(sbufs[b], u_sh.at[scvs[b]],
                                  sems[b]).wait()
            pltpu.make_async_copy(pvss[b], d_sh.at[scvs[b]],
                                  sems[b]).wait()

        _fire_io(0, 0)
        _fire_io(1, 1)
        plsc.subcore_barrier()
        _wait_io(0)
        _fire_g(0)
        _wait_io(1)
        _fire_g(1)

        def _step(b, ch, it):
            _wait_g(b)

            @pl.when(ch >= 2)
            def _():
                _wait_scat(b)

            _copy_scatter_idx(b)

            @pl.when(ch + 2 < _NCH)
            def _():
                _fire_io(b, ch + 2)

            _scale(b)
            _fire_scat(b)

            @pl.when(ch + 2 < _NCH)
            def _():
                _wait_io(b)
                _fire_g(b)

        def _pair(it, carry):
            ch2 = it * 2
            _step(0, ch2, it)
            _step(1, ch2 + 1, it)
            return carry
        lax.fori_loop(0, (_NCH - 1) // 2, _pair, 0)

        _wait_g(0)
        _wait_scat(0)
        _copy_scatter_idx(0)
        _scale(0)
        _fire_scat(0)
        _wait_scat(1)
        _wait_scat(0)

        plsc.subcore_barrier()

        r0 = s * 640
        pltpu.sync_copy(u_sh.at[pl.ds(r0, 640)],
                        u_out.at[h, c, pl.ds(r0, 640)])

        @pl.when(s < 10)
        def _():
            pltpu.sync_copy(
                d_sh.at[pl.ds(s * 1024, 1024)],
                d_out.at[pl.ds((h * _NC) * _NPAD + c * _NPAD + s * 1024, 1024)])

        plsc.subcore_barrier()


def _aggregate(ht0, ht1, ht2, src, dst, pv):
    mesh = plsc.VectorSubcoreMesh(core_axis_name="c", subcore_axis_name="s")
    fn = functools.partial(
        pl.kernel,
        out_type=(
            jax.ShapeDtypeStruct((_HEADS, _NC, _NPAD, _D), jnp.float32),
            jax.ShapeDtypeStruct((_HEADS * _NC * _NPAD,), jnp.float32),
        ),
        mesh=mesh,
        scratch_types=[
            pltpu.VMEM_SHARED((_NPAD, _D), jnp.float32),
            pltpu.VMEM_SHARED((_NPAD,), jnp.float32),
            pltpu.VMEM((_K, _D), jnp.float32),
            pltpu.VMEM((_K, _D), jnp.float32),
            pltpu.VMEM((_K, _D), jnp.float32),
            pltpu.VMEM((_K, _D), jnp.float32),
            pltpu.VMEM((_K,), jnp.int32),
            pltpu.VMEM((_K,), jnp.int32),
            pltpu.VMEM((_K,), jnp.int32),
            pltpu.VMEM((_K,), jnp.int32),
            pltpu.VMEM((_K,), jnp.int32),
            pltpu.VMEM((_K,), jnp.int32),
            pltpu.VMEM((_K,), jnp.float32),
            pltpu.VMEM((_K,), jnp.float32),
            pltpu.VMEM((_K,), jnp.float32),
            pltpu.VMEM((_K,), jnp.float32),
            pltpu.VMEM((1024,), jnp.float32),
            pltpu.SemaphoreType.DMA,
            pltpu.SemaphoreType.DMA,
            pltpu.SemaphoreType.DMA,
            pltpu.SemaphoreType.DMA,
            pltpu.SemaphoreType.DMA,
            pltpu.SemaphoreType.DMA,
        ],
        compiler_params=pltpu.CompilerParams(needs_layout_passes=False),
    )(_agg_body)
    return fn(ht0, ht1, ht2, src, dst, pv)


def _comb_body(u_ref, d_ref, b_ref, o_ref):
    i = pl.program_id(0)
    bn = o_ref.shape[0]
    acc = None
    for h in range(_HEADS):
        u = u_ref[h, 0] + u_ref[h, 1]
        d = (d_ref[h, 0, pl.ds(i * bn, bn)] + d_ref[h, 1, pl.ds(i * bn, bn)])
        t = u / (d + 1e-16)[:, None]
        acc = t if acc is None else acc + t
    o_ref[...] = acc * (1.0 / _HEADS) + b_ref[...][None, :]


def _combine(u, d, bias):
    bn = 1024
    grid = _NPAD // bn
    return pl.pallas_call(
        _comb_body,
        grid=(grid,),
        in_specs=[
            pl.BlockSpec((_HEADS, _NC, bn, _D), lambda i: (0, 0, i, 0)),
            pl.BlockSpec((_HEADS, _NC, _NPAD), lambda i: (0, 0, 0)),
            pl.BlockSpec((_D,), lambda i: (0,)),
        ],
        out_specs=pl.BlockSpec((bn, _D), lambda i: (i, 0)),
        out_shape=jax.ShapeDtypeStruct((_NPAD, _D), jnp.float32),
    )(u, d, bias)


def kernel(H, edge_index, W, a, bias):
    h_pad = jnp.zeros((_NPAD, _D), H.dtype).at[:_N].set(H)
    ht0, ht1, ht2, s1, s2 = _project(h_pad, W, a)
    src = edge_index[0]
    dst = edge_index[1]
    pv = _score(s1, s2, src, dst)
    u, d = _aggregate(ht0, ht1, ht2, src, dst, pv)
    out = _combine(u, d.reshape(_HEADS, _NC, _NPAD), bias)
    return out[:_N]


# speedup vs baseline: 1.1884x; 1.1884x over previous
import functools

import numpy as np

import jax
import jax.numpy as jnp
from jax import lax
from jax.experimental import pallas as pl
from jax.experimental.pallas import tpu as pltpu
from jax.experimental.pallas import tpu_sc as plsc

_N = 10000
_E = 320000
_D = 128
_HEADS = 3
_ALPHA = 0.2

_NC = 2
_NS = 16
_K = 80
_EPW = _E // (_NC * _NS)
_NCH = _EPW // _K
_NPAD = 10240
_WITH_P = True


def _proj_body(h_ref, w_ref, a_ref, ht0_ref, ht1_ref, ht2_ref, s1_ref, s2_ref):
    hb = h_ref[...]
    ht_refs = (ht0_ref, ht1_ref, ht2_ref)
    for h in range(_HEADS):
        w = w_ref[h]
        ht = lax.dot_general(hb, w, (((1,), (1,)), ((), ())),
                             preferred_element_type=jnp.float32)
        htb = ht.astype(jnp.bfloat16)
        lo = lax.bitcast_convert_type(htb[:, :_D // 2],
                                      jnp.uint16).astype(jnp.uint32)
        hi = lax.bitcast_convert_type(htb[:, _D // 2:],
                                      jnp.uint16).astype(jnp.uint32)
        ht_refs[h][...] = lax.bitcast_convert_type((hi << 16) | lo, jnp.int32)
        s1_ref[h, 0] = jnp.dot(ht, a_ref[h, :_D],
                               preferred_element_type=jnp.float32)
        s2_ref[h, 0] = jnp.dot(ht, a_ref[h, _D:],
                               preferred_element_type=jnp.float32)


def _project(H, W, a):
    bn = 1024
    grid = _NPAD // bn
    out_shape = (
        jax.ShapeDtypeStruct((_NPAD, _D // 2), jnp.int32),
        jax.ShapeDtypeStruct((_NPAD, _D // 2), jnp.int32),
        jax.ShapeDtypeStruct((_NPAD, _D // 2), jnp.int32),
        jax.ShapeDtypeStruct((_HEADS, 1, _NPAD), jnp.float32),
        jax.ShapeDtypeStruct((_HEADS, 1, _NPAD), jnp.float32),
    )
    return pl.pallas_call(
        _proj_body,
        grid=(grid,),
        in_specs=[
            pl.BlockSpec((bn, _D), lambda i: (i, 0)),
            pl.BlockSpec((_HEADS, _D, _D), lambda i: (0, 0, 0)),
            pl.BlockSpec((_HEADS, 2 * _D), lambda i: (0, 0)),
        ],
        out_specs=(
            pl.BlockSpec((bn, _D // 2), lambda i: (i, 0)),
            pl.BlockSpec((bn, _D // 2), lambda i: (i, 0)),
            pl.BlockSpec((bn, _D // 2), lambda i: (i, 0)),
            pl.BlockSpec((_HEADS, 1, bn), lambda i: (0, 0, i)),
            pl.BlockSpec((_HEADS, 1, bn), lambda i: (0, 0, i)),
        ),
        out_shape=out_shape,
    )(H, W, a)


def _score_body(s1h, s2h, srch, dsth, pv_out,
                s1v, s2v, src_all, dst_all, pv_all):
    c = lax.axis_index("c")
    s = lax.axis_index("s")
    base_e = c * (_E // _NC) + s * _EPW
    pltpu.sync_copy(srch.at[pl.ds(base_e, _EPW)], src_all)
    pltpu.sync_copy(dsth.at[pl.ds(base_e, _EPW)], dst_all)
    for h in range(_HEADS):
        pltpu.sync_copy(s1h.at[h, 0], s1v)
        pltpu.sync_copy(s2h.at[h, 0], s2v)

        @plsc.parallel_loop(0, _EPW // 16, unroll=4)
        def _(g):
            sl = pl.ds(g * 16, 16)
            e = (plsc.load_gather(s1v, [src_all[sl]])
                 + plsc.load_gather(s2v, [dst_all[sl]]))
            e = jnp.where(e > 0, e, _ALPHA * e)
            pv_all[sl] = jnp.exp(e)

        pltpu.sync_copy(pv_all, pv_out.at[pl.ds(h * _E + base_e, _EPW)])


def _score(s1, s2, src, dst):
    mesh = plsc.VectorSubcoreMesh(core_axis_name="c", subcore_axis_name="s")
    fn = functools.partial(
        pl.kernel,
        out_type=jax.ShapeDtypeStruct((_HEADS * _E,), jnp.float32),
        mesh=mesh,
        scratch_types=[
            pltpu.VMEM((_NPAD,), jnp.float32),
            pltpu.VMEM((_NPAD,), jnp.float32),
            pltpu.VMEM((_EPW,), jnp.int32),
            pltpu.VMEM((_EPW,), jnp.int32),
            pltpu.VMEM((_EPW,), jnp.float32),
        ],
        compiler_params=pltpu.CompilerParams(needs_layout_passes=False),
    )(_score_body)
    return fn(s1, s2, src, dst)


def _agg_body(ht0, ht1, ht2, srch, dsth, pvh, u_out, d_out,
              u_sh, d_sh, gbuf0, gbuf1, sbuf0, sbuf1,
              srcv0, srcv1, dstv0, dstv1, scv0, scv1,
              pvb0, pvb1, pvs0, pvs1, zvec,
              semio0, semio1, semg0, semg1, sems0, sems1):
    gbufs = (gbuf0, gbuf1)
    sbufs = (sbuf0, sbuf1)
    srcvs = (srcv0, srcv1)
    dstvs = (dstv0, dstv1)
    scvs = (scv0, scv1)
    pvbs = (pvb0, pvb1)
    pvss = (pvs0, pvs1)
    semio = (semio0, semio1)
    semg = (semg0, semg1)
    sems = (sems0, sems1)
    c = lax.axis_index("c")
    s = lax.axis_index("s")
    z16 = jnp.zeros((16,), jnp.float32)
    base_e = c * (_E // _NC) + s * _EPW

    def _zv(i, carry):
        zvec[pl.ds(i * 16, 16)] = z16
        return carry
    lax.fori_loop(0, zvec.shape[0] // 16, _zv, 0)

    ht_hbms = (ht0, ht1, ht2)
    for h in range(_HEADS):
        ht_h = ht_hbms[h]

        def _zg(i, carry):
            for s8 in range(8):
                sbuf0[i, pl.ds(s8 * 16, 16)] = z16
            return carry
        lax.fori_loop(0, _K, _zg, 0)
        for j in range(8):
            pltpu.sync_copy(sbuf0, u_sh.at[pl.ds(s * 640 + j * _K, _K)])

        @pl.when(s < 10)
        def _():
            pltpu.sync_copy(zvec, d_sh.at[pl.ds(s * 1024, 1024)])

        def _fire_io(b, ch):
            off = base_e + ch * _K
            pltpu.async_copy(srch.at[pl.ds(off, _K)], srcvs[b], semio[b])
            pltpu.async_copy(dsth.at[pl.ds(off, _K)], dstvs[b], semio[b])
            pltpu.async_copy(pvh.at[pl.ds(h * _E + off, _K)], pvbs[b],
                             semio[b])

        def _wait_io(b):
            pltpu.make_async_copy(srch.at[pl.ds(0, _K)], srcvs[b],
                                  semio[b]).wait()
            pltpu.make_async_copy(dsth.at[pl.ds(0, _K)], dstvs[b],
                                  semio[b]).wait()
            pltpu.make_async_copy(pvh.at[pl.ds(0, _K)], pvbs[b],
                                  semio[b]).wait()

        def _fire_g(b):
            pltpu.async_copy(ht_h.at[dstvs[b]], gbufs[b], semg[b])

        def _wait_g(b):
            pltpu.make_async_copy(ht_h.at[dstvs[b]], gbufs[b],
                                  semg[b]).wait()

        def _scale(b):
            gbuf, sbuf, pvb = gbufs[b], sbufs[b], pvss[b]

            @plsc.parallel_loop(0, _K, unroll=2)
            def _(i):
                pb = plsc.load_gather(pvb, [jnp.full((16,), i, jnp.int32)])
                for g in range(4):
                    v = plsc.bitcast(gbuf[i, pl.ds(g * 16, 16)], jnp.bfloat16)
                    lo, hi = plsc.unpack(v, format=plsc.PackFormat.INTERLEAVED)
                    sbuf[i, pl.ds(g * 16, 16)] = lo * pb
                    sbuf[i, pl.ds(g * 16 + 64, 16)] = hi * pb

        def _copy_scatter_idx(b):
            for g in range(_K // 16):
                sl = pl.ds(g * 16, 16)
                scvs[b][sl] = srcvs[b][sl]
                pvss[b][sl] = pvbs[b][sl]

        def _fire_scat(b):
            pltpu.async_copy(sbufs[b], u_sh.at[scvs[b]], sems[b], add=True)
            if _WITH_P:
                pltpu.async_copy(pvss[b], d_sh.at[scvs[b]], sems[b], add=True)

        def _wait_scat(b):
            pltpu.make_async_copy(sbufs[b], u_sh.at[scvs[b]],
                                  sems[b]).wait()
            if _WITH_P:
                pltpu.make_async_copy(pvss[b], d_sh.at[scvs[b]],
                                      sems[b]).wait()

        _fire_io(0, 0)
        _fire_io(1, 1)
        plsc.subcore_barrier()
        _wait_io(0)
        _fire_g(0)
        _wait_io(1)
        _fire_g(1)

        def _step(b, ch, it):
            _wait_g(b)

            @pl.when(ch >= 2)
            def _():
                _wait_scat(b)

            _copy_scatter_idx(b)

            @pl.when(ch + 2 < _NCH)
            def _():
                _fire_io(b, ch + 2)

            _scale(b)
            _fire_scat(b)

            @pl.when(ch + 2 < _NCH)
            def _():
                _wait_io(b)
                _fire_g(b)

        def _pair(it, carry):
            ch2 = it * 2
            _step(0, ch2, it)
            _step(1, ch2 + 1, it)
            return carry
        lax.fori_loop(0, (_NCH - 1) // 2, _pair, 0)

        _wait_g(0)
        _wait_scat(0)
        _copy_scatter_idx(0)
        _scale(0)
        _fire_scat(0)
        _wait_scat(1)
        _wait_scat(0)

        plsc.subcore_barrier()

        r0 = s * 640
        pltpu.sync_copy(u_sh.at[pl.ds(r0, 640)],
                        u_out.at[h, c, pl.ds(r0, 640)])

        @pl.when(s < 10)
        def _():
            pltpu.sync_copy(
                d_sh.at[pl.ds(s * 1024, 1024)],
                d_out.at[pl.ds((h * _NC) * _NPAD + c * _NPAD + s * 1024, 1024)])

        plsc.subcore_barrier()


def _aggregate(ht0, ht1, ht2, src, dst, pv):
    mesh = plsc.VectorSubcoreMesh(core_axis_name="c", subcore_axis_name="s")
    fn = functools.partial(
        pl.kernel,
        out_type=(
            jax.ShapeDtypeStruct((_HEADS, _NC, _NPAD, _D), jnp.float32),
            jax.ShapeDtypeStruct((_HEADS * _NC * _NPAD,), jnp.float32),
        ),
        mesh=mesh,
        scratch_types=[
            pltpu.VMEM_SHARED((_NPAD, _D), jnp.float32),
            pltpu.VMEM_SHARED((_NPAD,), jnp.float32),
            pltpu.VMEM((_K, _D // 2), jnp.int32),
            pltpu.VMEM((_K, _D // 2), jnp.int32),
            pltpu.VMEM((_K, _D), jnp.float32),
            pltpu.VMEM((_K, _D), jnp.float32),
            pltpu.VMEM((_K,), jnp.int32),
            pltpu.VMEM((_K,), jnp.int32),
            pltpu.VMEM((_K,), jnp.int32),
            pltpu.VMEM((_K,), jnp.int32),
            pltpu.VMEM((_K,), jnp.int32),
            pltpu.VMEM((_K,), jnp.int32),
            pltpu.VMEM((_K,), jnp.float32),
            pltpu.VMEM((_K,), jnp.float32),
            pltpu.VMEM((_K,), jnp.float32),
            pltpu.VMEM((_K,), jnp.float32),
            pltpu.VMEM((1024,), jnp.float32),
            pltpu.SemaphoreType.DMA,
            pltpu.SemaphoreType.DMA,
            pltpu.SemaphoreType.DMA,
            pltpu.SemaphoreType.DMA,
            pltpu.SemaphoreType.DMA,
            pltpu.SemaphoreType.DMA,
        ],
        compiler_params=pltpu.CompilerParams(needs_layout_passes=False,
                                             use_tc_tiling_on_sc=False),
    )(_agg_body)
    return fn(ht0, ht1, ht2, src, dst, pv)


def _comb_body(u_ref, d_ref, b_ref, o_ref):
    i = pl.program_id(0)
    bn = o_ref.shape[0]
    acc = None
    for h in range(_HEADS):
        u = u_ref[h, 0] + u_ref[h, 1]
        d = (d_ref[h, 0, pl.ds(i * bn, bn)] + d_ref[h, 1, pl.ds(i * bn, bn)])
        t = u / (d + 1e-16)[:, None]
        acc = t if acc is None else acc + t
    o_ref[...] = acc * (1.0 / _HEADS) + b_ref[...][None, :]


def _combine(u, d, bias):
    bn = 1024
    grid = _NPAD // bn
    return pl.pallas_call(
        _comb_body,
        grid=(grid,),
        in_specs=[
            pl.BlockSpec((_HEADS, _NC, bn, _D), lambda i: (0, 0, i, 0)),
            pl.BlockSpec((_HEADS, _NC, _NPAD), lambda i: (0, 0, 0)),
            pl.BlockSpec((_D,), lambda i: (0,)),
        ],
        out_specs=pl.BlockSpec((bn, _D), lambda i: (i, 0)),
        out_shape=jax.ShapeDtypeStruct((_NPAD, _D), jnp.float32),
    )(u, d, bias)


def kernel(H, edge_index, W, a, bias):
    h_pad = jnp.zeros((_NPAD, _D), H.dtype).at[:_N].set(H)
    ht0, ht1, ht2, s1, s2 = _project(h_pad, W, a)
    src = edge_index[0]
    dst = edge_index[1]
    pv = _score(s1, s2, src, dst)
    u, d = _aggregate(ht0, ht1, ht2, src, dst, pv)
    out = _combine(u, d.reshape(_HEADS, _NC, _NPAD), bias)
    return out[:_N]
